# baseline (speedup 1.0000x reference)
import functools

import jax
import jax.numpy as jnp
from jax import lax
from jax.experimental import pallas as pl
from jax.experimental.pallas import tpu as pltpu
from jax.experimental.pallas import tpu_sc as plsc

TEMPERATURE = 0.1

_CH = 64
_LANES = 16


def _sc_pool_body(ch_ref, params_ref, out_ref, pvec_ref, buf_ref,
                  acc_ref, *, L, H, NSEG, SEGS_PER_W, NC):
    cid = lax.axis_index("c")
    sid = lax.axis_index("s")
    wid = sid * NC + cid
    HREG = H // _LANES
    zero = jnp.zeros((_LANES,), jnp.float32)
    for i in range(SEGS_PER_W):
        g = wid * SEGS_PER_W + i
        pltpu.sync_copy(params_ref.at[g], pvec_ref)
        v = pvec_ref[...]
        bb = v[0]
        ss = v[1]
        nch = v[2]
        lim = v[3]
        chunk0 = v[4]
        for h in range(HREG):
            acc_ref[pl.ds(h * _LANES, _LANES)] = zero

        def chunk_body(c, carry):
            row0c = (chunk0 + c) * _CH
            pltpu.sync_copy(ch_ref.at[bb, pl.ds(row0c, _CH), :], buf_ref)

            def row_body(j, rc):
                r = row0c + j

                @pl.when((r >= ss) & (r <= lim))
                def _():
                    for h in range(HREG):
                        sl = pl.ds(h * _LANES, _LANES)
                        acc_ref[sl] += buf_ref[j, sl]

                return rc

            return lax.fori_loop(0, _CH, row_body, carry)

        lax.fori_loop(0, nch, chunk_body, 0)
        pltpu.sync_copy(acc_ref, out_ref.at[g])


def _loss_kernel(cc_ref, codec_ref, c2c_ref, nb_ref, ns_ref, pooled_ref,
                 sall_ref, lall_ref, out_ref, *, B, C, K, N):
    eps = jnp.float32(1e-12)
    cc = cc_ref[...]
    cc = cc / jnp.maximum(
        jnp.sqrt(jnp.sum(cc * cc, axis=1, keepdims=True)), eps)
    codec = codec_ref[...]
    codec = codec / jnp.maximum(
        jnp.sqrt(jnp.sum(codec * codec, axis=1, keepdims=True)), eps)

    c2c = c2c_ref[0, 0, :]
    c2c_cl = jnp.clip(c2c, 0, N - 1)
    jj = jax.lax.broadcasted_iota(jnp.int32, (N, N), 1)
    sel_pos = (jj == c2c_cl[:, None]).astype(jnp.float32)
    code_cent = jnp.dot(sel_pos, codec, preferred_element_type=jnp.float32)
    pos_sim = jnp.sum(cc * code_cent, axis=1)

    cnt = jnp.maximum(
        lall_ref[0, 0, :] - sall_ref[0, 0, :] + 1, 0
    ).astype(jnp.float32)
    pooled = pooled_ref[...]
    pooled = pooled / jnp.maximum(cnt, 1.0)[:, None]
    pooled = pooled / jnp.maximum(
        jnp.sqrt(jnp.sum(pooled * pooled, axis=1, keepdims=True)), eps)
    S = jnp.dot(cc, pooled.T, preferred_element_type=jnp.float32)

    nb = nb_ref[0, :, :]
    ns = ns_ref[0, :, :]
    in_range = (nb < B) & (ns < C)
    j = jnp.clip(nb, 0, B - 1) * C + jnp.clip(ns, 0, C - 1)
    jk = jax.lax.broadcasted_iota(jnp.int32, (N, K, N), 2)
    sel = (jk == j[:, :, None]).astype(jnp.float32)
    E = jnp.sum(S[:, None, :] * sel, axis=2)
    cnt_pos = (cnt > 0.0).astype(jnp.float32)
    neg_has = jnp.sum(cnt_pos[None, None, :] * sel, axis=2) > 0.0
    vmask = in_range & neg_has

    neg_exp = jnp.exp(E / TEMPERATURE)
    neg_sum = jnp.sum(jnp.where(vmask, neg_exp, 0.0), axis=1)
    pos_exp = jnp.exp(pos_sim / TEMPERATURE)
    lv = -jnp.log(pos_exp / (pos_exp + neg_sum + 1e-08))
    valid = (c2c < N) & jnp.any(vmask, axis=1)
    vals = jnp.where(valid, lv, 0.0)
    total = jnp.sum(vals)
    n = jnp.sum(valid.astype(jnp.float32))
    res = jnp.where(n > 0.0, total / jnp.maximum(n, 1.0), 0.0)
    out_ref[...] = jnp.reshape(res, (1, 1))


@jax.jit
def kernel(comment_centers, code_centers, all_code_centers,
           comment_to_code_map, negative_sample_indices, nl_hidden,
           code_hidden, total_code_tokens_list, valid_code_spans_batch,
           valid_comment_spans_batch, step_descriptions_batch):
    del all_code_centers, nl_hidden, valid_comment_spans_batch
    del step_descriptions_batch
    B, L, H = code_hidden.shape
    N, _ = comment_centers.shape
    _, C, K, _ = negative_sample_indices.shape

    spans = valid_code_spans_batch.astype(jnp.int32)
    starts2 = spans[:, :, 1, 0]
    totals = total_code_tokens_list.astype(jnp.int32)
    lims2 = jnp.minimum(spans[:, :, 1, 1], totals[:, None])

    sflat = starts2.reshape(N)
    lflat = lims2.reshape(N)
    bflat = jnp.repeat(jnp.arange(B, dtype=jnp.int32), C)
    cntf = jnp.maximum(lflat - sflat + 1, 0)
    chunk0 = sflat // _CH
    nchf = jnp.where(cntf > 0, lflat // _CH - chunk0 + 1, 0)
    params = jnp.zeros((N, _LANES), jnp.int32)
    params = params.at[:, 0].set(bflat)
    params = params.at[:, 1].set(sflat)
    params = params.at[:, 2].set(nchf)
    params = params.at[:, 3].set(lflat)
    params = params.at[:, 4].set(chunk0)

    info = plsc.get_sparse_core_info()
    NC, NS = info.num_cores, info.num_subcores
    NW = NC * NS
    SEGS_PER_W = (N + NW - 1) // NW
    mesh = plsc.VectorSubcoreMesh(core_axis_name="c", subcore_axis_name="s")

    sc_pool = pl.kernel(
        functools.partial(_sc_pool_body, L=L, H=H, NSEG=N,
                          SEGS_PER_W=SEGS_PER_W, NC=NC),
        mesh=mesh,
        out_type=jax.ShapeDtypeStruct((N, H), jnp.float32),
        scratch_types=[
            pltpu.VMEM((_LANES,), jnp.int32),
            pltpu.VMEM((_CH, H), jnp.float32),
            pltpu.VMEM((H,), jnp.float32),
        ],
    )
    pooled = sc_pool(code_hidden, params)

    negs = negative_sample_indices.astype(jnp.int32).reshape(N, K, 2)
    nb = negs[:, :, 0].reshape(1, N, K)
    ns = negs[:, :, 1].reshape(1, N, K)
    c2c = comment_to_code_map.astype(jnp.int32).reshape(1, 1, N)

    out = pl.pallas_call(
        functools.partial(_loss_kernel, B=B, C=C, K=K, N=N),
        in_specs=[
            pl.BlockSpec((N, H), lambda: (0, 0)),
            pl.BlockSpec((N, H), lambda: (0, 0)),
            pl.BlockSpec((1, 1, N), lambda: (0, 0, 0)),
            pl.BlockSpec((1, N, K), lambda: (0, 0, 0)),
            pl.BlockSpec((1, N, K), lambda: (0, 0, 0)),
            pl.BlockSpec((N, H), lambda: (0, 0)),
            pl.BlockSpec((1, 1, N), lambda: (0, 0, 0)),
            pl.BlockSpec((1, 1, N), lambda: (0, 0, 0)),
        ],
        out_specs=pl.BlockSpec((1, 1), lambda: (0, 0)),
        out_shape=jax.ShapeDtypeStruct((1, 1), jnp.float32),
    )(comment_centers, code_centers, c2c, nb, ns, pooled,
      sflat.reshape(1, 1, N), lflat.reshape(1, 1, N))

    return out[0, 0]


# speedup vs baseline: 2.8823x; 2.8823x over previous
import functools

import jax
import jax.numpy as jnp
from jax import lax
from jax.experimental import pallas as pl
from jax.experimental.pallas import tpu as pltpu
from jax.experimental.pallas import tpu_sc as plsc

TEMPERATURE = 0.1

_CH = 64
_LANES = 16


def _sc_pool_body(ch_ref, params_ref, out_ref, pvec_ref, buf_ref,
                  acc_ref, *, L, H, NSEG, SEGS_PER_W, NC):
    cid = lax.axis_index("c")
    sid = lax.axis_index("s")
    wid = sid * NC + cid
    HREG = H // _LANES
    zero = jnp.zeros((_LANES,), jnp.float32)
    for i in range(SEGS_PER_W):
        g = wid * SEGS_PER_W + i
        pltpu.sync_copy(params_ref.at[g], pvec_ref)
        v = pvec_ref[...]
        bb = v[0]
        ss = v[1]
        nch = v[2]
        lim = v[3]
        chunk0 = v[4]
        for h in range(HREG):
            acc_ref[pl.ds(h * _LANES, _LANES)] = zero

        NPASS = 4
        RPP = HREG // NPASS

        def chunk_body(c, carry):
            row0c = (chunk0 + c) * _CH
            pltpu.sync_copy(ch_ref.at[bb, pl.ds(row0c, _CH), :], buf_ref)
            is_full = (row0c >= ss) & (row0c + _CH - 1 <= lim)

            @pl.when(is_full)
            def _fast():
                for p in range(NPASS):
                    sls = [pl.ds((p * RPP + h) * _LANES, _LANES)
                           for h in range(RPP)]

                    def jbody(j, regs, sls=sls):
                        return tuple(regs[h] + buf_ref[j, sls[h]]
                                     for h in range(RPP))

                    regs = lax.fori_loop(
                        0, _CH, jbody,
                        tuple(acc_ref[sl] for sl in sls))
                    for h in range(RPP):
                        acc_ref[sls[h]] = regs[h]

            @pl.when(jnp.logical_not(is_full))
            def _slow():
                def row_body(j, rc):
                    r = row0c + j

                    @pl.when((r >= ss) & (r <= lim))
                    def _():
                        for h in range(HREG):
                            sl = pl.ds(h * _LANES, _LANES)
                            acc_ref[sl] += buf_ref[j, sl]

                    return rc

                lax.fori_loop(0, _CH, row_body, 0)

            return carry

        lax.fori_loop(0, nch, chunk_body, 0)
        pltpu.sync_copy(acc_ref, out_ref.at[g])


def _loss_kernel(cc_ref, codec_ref, c2c_ref, nb_ref, ns_ref, pooled_ref,
                 sall_ref, lall_ref, out_ref, *, B, C, K, N):
    eps = jnp.float32(1e-12)
    cc = cc_ref[...]
    cc = cc / jnp.maximum(
        jnp.sqrt(jnp.sum(cc * cc, axis=1, keepdims=True)), eps)
    codec = codec_ref[...]
    codec = codec / jnp.maximum(
        jnp.sqrt(jnp.sum(codec * codec, axis=1, keepdims=True)), eps)

    c2c = c2c_ref[0, 0, :]
    c2c_cl = jnp.clip(c2c, 0, N - 1)
    jj = jax.lax.broadcasted_iota(jnp.int32, (N, N), 1)
    sel_pos = (jj == c2c_cl[:, None]).astype(jnp.float32)
    code_cent = jnp.dot(sel_pos, codec, preferred_element_type=jnp.float32)
    pos_sim = jnp.sum(cc * code_cent, axis=1)

    cnt = jnp.maximum(
        lall_ref[0, 0, :] - sall_ref[0, 0, :] + 1, 0
    ).astype(jnp.float32)
    pooled = pooled_ref[...]
    pooled = pooled / jnp.maximum(cnt, 1.0)[:, None]
    pooled = pooled / jnp.maximum(
        jnp.sqrt(jnp.sum(pooled * pooled, axis=1, keepdims=True)), eps)
    S = jnp.dot(cc, pooled.T, preferred_element_type=jnp.float32)

    nb = nb_ref[0, :, :]
    ns = ns_ref[0, :, :]
    in_range = (nb < B) & (ns < C)
    j = jnp.clip(nb, 0, B - 1) * C + jnp.clip(ns, 0, C - 1)
    jk = jax.lax.broadcasted_iota(jnp.int32, (N, K, N), 2)
    sel = (jk == j[:, :, None]).astype(jnp.float32)
    E = jnp.sum(S[:, None, :] * sel, axis=2)
    cnt_pos = (cnt > 0.0).astype(jnp.float32)
    neg_has = jnp.sum(cnt_pos[None, None, :] * sel, axis=2) > 0.0
    vmask = in_range & neg_has

    neg_exp = jnp.exp(E / TEMPERATURE)
    neg_sum = jnp.sum(jnp.where(vmask, neg_exp, 0.0), axis=1)
    pos_exp = jnp.exp(pos_sim / TEMPERATURE)
    lv = -jnp.log(pos_exp / (pos_exp + neg_sum + 1e-08))
    valid = (c2c < N) & jnp.any(vmask, axis=1)
    vals = jnp.where(valid, lv, 0.0)
    total = jnp.sum(vals)
    n = jnp.sum(valid.astype(jnp.float32))
    res = jnp.where(n > 0.0, total / jnp.maximum(n, 1.0), 0.0)
    out_ref[...] = jnp.reshape(res, (1, 1))


@jax.jit
def kernel(comment_centers, code_centers, all_code_centers,
           comment_to_code_map, negative_sample_indices, nl_hidden,
           code_hidden, total_code_tokens_list, valid_code_spans_batch,
           valid_comment_spans_batch, step_descriptions_batch):
    del all_code_centers, nl_hidden, valid_comment_spans_batch
    del step_descriptions_batch
    B, L, H = code_hidden.shape
    N, _ = comment_centers.shape
    _, C, K, _ = negative_sample_indices.shape

    spans = valid_code_spans_batch.astype(jnp.int32)
    starts2 = spans[:, :, 1, 0]
    totals = total_code_tokens_list.astype(jnp.int32)
    lims2 = jnp.minimum(spans[:, :, 1, 1], totals[:, None])

    sflat = starts2.reshape(N)
    lflat = lims2.reshape(N)
    bflat = jnp.repeat(jnp.arange(B, dtype=jnp.int32), C)
    cntf = jnp.maximum(lflat - sflat + 1, 0)
    chunk0 = sflat // _CH
    nchf = jnp.where(cntf > 0, lflat // _CH - chunk0 + 1, 0)
    params = jnp.zeros((N, _LANES), jnp.int32)
    params = params.at[:, 0].set(bflat)
    params = params.at[:, 1].set(sflat)
    params = params.at[:, 2].set(nchf)
    params = params.at[:, 3].set(lflat)
    params = params.at[:, 4].set(chunk0)

    info = plsc.get_sparse_core_info()
    NC, NS = info.num_cores, info.num_subcores
    NW = NC * NS
    SEGS_PER_W = (N + NW - 1) // NW
    mesh = plsc.VectorSubcoreMesh(core_axis_name="c", subcore_axis_name="s")

    sc_pool = pl.kernel(
        functools.partial(_sc_pool_body, L=L, H=H, NSEG=N,
                          SEGS_PER_W=SEGS_PER_W, NC=NC),
        mesh=mesh,
        out_type=jax.ShapeDtypeStruct((N, H), jnp.float32),
        scratch_types=[
            pltpu.VMEM((_LANES,), jnp.int32),
            pltpu.VMEM((_CH, H), jnp.float32),
            pltpu.VMEM((H,), jnp.float32),
        ],
    )
    pooled = sc_pool(code_hidden, params)

    negs = negative_sample_indices.astype(jnp.int32).reshape(N, K, 2)
    nb = negs[:, :, 0].reshape(1, N, K)
    ns = negs[:, :, 1].reshape(1, N, K)
    c2c = comment_to_code_map.astype(jnp.int32).reshape(1, 1, N)

    out = pl.pallas_call(
        functools.partial(_loss_kernel, B=B, C=C, K=K, N=N),
        in_specs=[
            pl.BlockSpec((N, H), lambda: (0, 0)),
            pl.BlockSpec((N, H), lambda: (0, 0)),
            pl.BlockSpec((1, 1, N), lambda: (0, 0, 0)),
            pl.BlockSpec((1, N, K), lambda: (0, 0, 0)),
            pl.BlockSpec((1, N, K), lambda: (0, 0, 0)),
            pl.BlockSpec((N, H), lambda: (0, 0)),
            pl.BlockSpec((1, 1, N), lambda: (0, 0, 0)),
            pl.BlockSpec((1, 1, N), lambda: (0, 0, 0)),
        ],
        out_specs=pl.BlockSpec((1, 1), lambda: (0, 0)),
        out_shape=jax.ShapeDtypeStruct((1, 1), jnp.float32),
    )(comment_centers, code_centers, c2c, nb, ns, pooled,
      sflat.reshape(1, 1, N), lflat.reshape(1, 1, N))

    return out[0, 0]


# speedup vs baseline: 22.2708x; 7.7268x over previous
import functools

import jax
import jax.numpy as jnp
from jax.experimental import pallas as pl
from jax.experimental.pallas import tpu as pltpu

TEMPERATURE = 0.1


def _fused_kernel(starts_ref, lims_ref, ch_ref, cc_ref, codec_ref, c2c_ref,
                  nb_ref, ns_ref, sall_ref, lall_ref, out_ref, pooled_ref,
                  *, B, C, K, N, ROWS):
    g = pl.program_id(0)
    ng = pl.num_programs(0)
    RC = ROWS * C
    L = ch_ref.shape[1]
    s = starts_ref[0, 0, :]
    lim = lims_ref[0, 0, :]
    t = jax.lax.broadcasted_iota(jnp.int32, (RC, ROWS * L), 1)
    mask = (t >= s[:, None]) & (t <= lim[:, None])
    maskf = mask.astype(jnp.float32)
    ch = ch_ref[...].reshape(ROWS * L, ch_ref.shape[2])
    pooled_ref[pl.ds(g * RC, RC), :] = jnp.dot(
        maskf, ch, preferred_element_type=jnp.float32)

    @pl.when(g == ng - 1)
    def _epilogue():
        eps = jnp.float32(1e-12)
        cc = cc_ref[...]
        cc = cc / jnp.maximum(
            jnp.sqrt(jnp.sum(cc * cc, axis=1, keepdims=True)), eps)
        codec = codec_ref[...]
        codec = codec / jnp.maximum(
            jnp.sqrt(jnp.sum(codec * codec, axis=1, keepdims=True)), eps)

        c2c = c2c_ref[0, 0, :]
        c2c_cl = jnp.clip(c2c, 0, N - 1)
        jj = jax.lax.broadcasted_iota(jnp.int32, (N, N), 1)
        sel_pos = (jj == c2c_cl[:, None]).astype(jnp.float32)
        code_cent = jnp.dot(sel_pos, codec,
                            preferred_element_type=jnp.float32)
        pos_sim = jnp.sum(cc * code_cent, axis=1)

        cnt = jnp.maximum(
            lall_ref[0, 0, :] - sall_ref[0, 0, :] + 1, 0
        ).astype(jnp.float32)
        pooled = pooled_ref[...]
        pooled = pooled / jnp.maximum(cnt, 1.0)[:, None]
        pooled = pooled / jnp.maximum(
            jnp.sqrt(jnp.sum(pooled * pooled, axis=1, keepdims=True)), eps)
        S = jnp.dot(cc, pooled.T,
                    preferred_element_type=jnp.float32)

        nb = nb_ref[0, :, :]
        ns = ns_ref[0, :, :]
        in_range = (nb < B) & (ns < C)
        j = jnp.clip(nb, 0, B - 1) * C + jnp.clip(ns, 0, C - 1)
        jk = jax.lax.broadcasted_iota(jnp.int32, (N, K, N), 2)
        sel = (jk == j[:, :, None]).astype(jnp.float32)
        E = jnp.sum(S[:, None, :] * sel, axis=2)
        cnt_pos = (cnt > 0.0).astype(jnp.float32)
        neg_has = jnp.sum(cnt_pos[None, None, :] * sel, axis=2) > 0.0
        vmask = in_range & neg_has

        neg_exp = jnp.exp(E / TEMPERATURE)
        neg_sum = jnp.sum(jnp.where(vmask, neg_exp, 0.0), axis=1)
        pos_exp = jnp.exp(pos_sim / TEMPERATURE)
        lv = -jnp.log(pos_exp / (pos_exp + neg_sum + 1e-08))
        valid = (c2c < N) & jnp.any(vmask, axis=1)
        vals = jnp.where(valid, lv, 0.0)
        total = jnp.sum(vals)
        n = jnp.sum(valid.astype(jnp.float32))
        res = jnp.where(n > 0.0, total / jnp.maximum(n, 1.0), 0.0)
        out_ref[...] = jnp.reshape(res, (1, 1))


@jax.jit
def kernel(comment_centers, code_centers, all_code_centers,
           comment_to_code_map, negative_sample_indices, nl_hidden,
           code_hidden, total_code_tokens_list, valid_code_spans_batch,
           valid_comment_spans_batch, step_descriptions_batch):
    del all_code_centers, nl_hidden, valid_comment_spans_batch
    del step_descriptions_batch
    B, L, H = code_hidden.shape
    N, _ = comment_centers.shape
    _, C, K, _ = negative_sample_indices.shape

    spans = valid_code_spans_batch.astype(jnp.int32)
    starts2 = spans[:, :, 1, 0]
    totals = total_code_tokens_list.astype(jnp.int32)
    lims2 = jnp.minimum(spans[:, :, 1, 1], totals[:, None])

    ROWS = 1
    NG = B // ROWS
    RC = ROWS * C
    off = (jnp.arange(B, dtype=jnp.int32) % ROWS)[:, None] * L
    starts_g = (starts2 + off).reshape(NG, 1, RC)
    lims_g = (lims2 + off).reshape(NG, 1, RC)

    negs = negative_sample_indices.astype(jnp.int32).reshape(N, K, 2)
    nb = negs[:, :, 0].reshape(1, N, K)
    ns = negs[:, :, 1].reshape(1, N, K)
    c2c = comment_to_code_map.astype(jnp.int32).reshape(1, 1, N)

    out = pl.pallas_call(
        functools.partial(_fused_kernel, B=B, C=C, K=K, N=N, ROWS=ROWS),
        grid=(NG,),
        in_specs=[
            pl.BlockSpec((1, 1, RC), lambda g: (g, 0, 0)),
            pl.BlockSpec((1, 1, RC), lambda g: (g, 0, 0)),
            pl.BlockSpec((ROWS, L, H), lambda g: (g, 0, 0)),
            pl.BlockSpec((N, H), lambda g: (0, 0)),
            pl.BlockSpec((N, H), lambda g: (0, 0)),
            pl.BlockSpec((1, 1, N), lambda g: (0, 0, 0)),
            pl.BlockSpec((1, N, K), lambda g: (0, 0, 0)),
            pl.BlockSpec((1, N, K), lambda g: (0, 0, 0)),
            pl.BlockSpec((1, 1, N), lambda g: (0, 0, 0)),
            pl.BlockSpec((1, 1, N), lambda g: (0, 0, 0)),
        ],
        out_specs=pl.BlockSpec((1, 1), lambda g: (0, 0)),
        out_shape=jax.ShapeDtypeStruct((1, 1), jnp.float32),
        scratch_shapes=[
            pltpu.VMEM((N, H), jnp.float32),
        ],
    )(starts_g, lims_g, code_hidden, comment_centers, code_centers, c2c,
      nb, ns, starts2.reshape(1, 1, N), lims2.reshape(1, 1, N))

    return out[0, 0]
